# baseline (speedup 1.0000x reference)
import functools

import jax
import jax.numpy as jnp
from jax import lax
from jax.experimental import pallas as pl
from jax.experimental.pallas import tpu as pltpu
from jax.experimental.pallas import tpu_sc as plsc

_N = 16384
_D = 768
_NC = 2
_NS = 16
_NW = _NC * _NS
_RW = _N // _NW
_K = 64
_NG = _RW // 16
_NJ = _D // 16


def _sc_body(maski_hbm, docs_hbm, part_hbm, cnt_hbm,
             mask_v, idx_v, rows_v, acc_v, cnt_v, sem):
    wid = lax.axis_index("s") * _NC + lax.axis_index("c")
    base = wid * _RW
    pltpu.sync_copy(maski_hbm.at[pl.ds(base, _RW)], mask_v)

    zeros16 = jnp.zeros((16,), jnp.int32)
    for g in range(_NG):
        idx_v[pl.ds(g * 16, 16)] = zeros16

    off = jnp.int32(0)
    for g in range(_NG):
        mi = mask_v[pl.ds(g * 16, 16)]
        mb = mi > 0
        pos = off + plsc.cumsum(mi) - 1
        ids = base + g * 16 + lax.iota(jnp.int32, 16)
        plsc.store_scatter(idx_v, [pos], ids, mask=mb)
        off = off + jnp.sum(mi)
    c = off

    nch = (c + _K - 1) // _K

    def chunk_body(ch, acc):
        pltpu.async_copy(docs_hbm.at[idx_v.at[pl.ds(ch * _K, _K)]],
                         rows_v, sem).wait()
        nr = jnp.minimum(_K, c - ch * _K)

        def row_body(r, acc):
            return [acc[j] + rows_v[r, pl.ds(16 * j, 16)] for j in range(_NJ)]

        return lax.fori_loop(0, nr, row_body, acc)

    acc = lax.fori_loop(0, nch, chunk_body,
                        [jnp.zeros((16,), jnp.float32)] * _NJ)

    for j in range(_NJ):
        acc_v[pl.ds(16 * j, 16)] = acc[j]
    cnt_v[...] = jnp.zeros((16,), jnp.float32) + c.astype(jnp.float32)
    pltpu.sync_copy(acc_v, part_hbm.at[wid])
    pltpu.sync_copy(cnt_v, cnt_hbm.at[wid])


def _fin_body(p_ref, c_ref, q_ref, out_ref):
    s = jnp.sum(p_ref[...], axis=0, keepdims=True)
    cnt = jnp.sum(c_ref[...]) * (1.0 / 16.0)
    mean = s / jnp.maximum(cnt, 1.0)
    wa = (q_ref[...] + mean) * 0.5
    norm = jnp.maximum(jnp.sqrt(jnp.sum(wa * wa)), 1e-12)
    out_ref[...] = jnp.where(cnt == 0.0, q_ref[...], wa / norm)


@jax.jit
def kernel(question_embedding, document_embeddings, mask):
    maski = mask.astype(jnp.int32)
    mesh = plsc.VectorSubcoreMesh(core_axis_name="c", subcore_axis_name="s",
                                  num_cores=_NC, num_subcores=_NS)
    partials, counts = pl.kernel(
        _sc_body,
        out_type=[
            jax.ShapeDtypeStruct((_NW, _D), jnp.float32),
            jax.ShapeDtypeStruct((_NW, 16), jnp.float32),
        ],
        mesh=mesh,
        scratch_types=[
            pltpu.VMEM((_RW,), jnp.int32),
            pltpu.VMEM((_RW,), jnp.int32),
            pltpu.VMEM((_K, _D), jnp.float32),
            pltpu.VMEM((_D,), jnp.float32),
            pltpu.VMEM((16,), jnp.float32),
            pltpu.SemaphoreType.DMA,
        ],
        compiler_params=pltpu.CompilerParams(needs_layout_passes=False),
    )(maski, document_embeddings)

    q = question_embedding.reshape(1, _D)
    out = pl.pallas_call(
        _fin_body,
        out_shape=jax.ShapeDtypeStruct((1, _D), jnp.float32),
    )(partials, counts, q)
    return out.reshape(_D)
